# Initial kernel scaffold; baseline (speedup 1.0000x reference)
import jax
import jax.numpy as jnp
from jax.experimental import pallas as pl


def kernel(input_atom, input_bond, atom_nei_idx, bond_nei_idx, num_nbs, W_atom, W_U2, b_U2, W_U1, b_U1):
    raise NotImplementedError("write your pallas kernel here")


# speedup vs baseline: 22.3366x; 22.3366x over previous
import jax
import jax.numpy as jnp
from jax import lax
from jax.experimental import pallas as pl

_B, _N, _K, _FB = 76, 151, 10, 5
_H, _DEPTH = 128, 3
_AF = 89
_NP = 160
_KP = 16


def _tc_body(x_ref, bond_ref, idxa_ref, idxb_ref, WaT_ref, W2aT_ref,
             W2bT_ref, W1aT_ref, W1bT_ref, bU1_ref, out_ref):
    f32 = jnp.float32
    x = x_ref[0]
    A = jnp.dot(x, WaT_ref[...], preferred_element_type=f32)
    iota = lax.broadcasted_iota(jnp.int32, (_NP, _NP), 1)
    idxa = idxa_ref[0]
    idxb = idxb_ref[0]
    Ma = jnp.zeros((_NP, _NP), f32)
    Mb = jnp.zeros((_NP, _NP), f32)
    for k in range(_K):
        Ma = Ma + (idxa[:, k:k + 1] == iota).astype(f32)
        Mb = Mb + (idxb[:, k:k + 1] == iota).astype(f32)
    bond_sum = jnp.dot(Mb, bond_ref[0], preferred_element_type=f32)
    bondpart = jnp.dot(bond_sum, W2bT_ref[...], preferred_element_type=f32)
    for _ in range(_DEPTH):
        S = jnp.dot(Ma, A, preferred_element_type=f32)
        nei = jnp.dot(S, W2aT_ref[...], preferred_element_type=f32) + bondpart
        A = (jnp.dot(A, W1aT_ref[...], preferred_element_type=f32)
             + jnp.dot(nei, W1bT_ref[...], preferred_element_type=f32)
             + bU1_ref[...])
    out_ref[0] = A


def kernel(input_atom, input_bond, atom_nei_idx, bond_nei_idx, num_nbs,
           W_atom, W_U2, b_U2, W_U1, b_U1):
    f32 = jnp.float32
    xp = jnp.zeros((_B, _NP, _H), f32)
    xp = xp.at[:, :_N, :_AF].set(input_atom)
    bond_aug = jnp.zeros((_B, _NP, _H), f32)
    bond_aug = bond_aug.at[:, :_N, :_FB].set(input_bond)
    bond_aug = bond_aug.at[:, :_N, _FB].set(1.0)

    mask = jnp.arange(_K, dtype=jnp.int32)[None, None, :] < num_nbs[:, :, None]
    idxa = jnp.where(mask, atom_nei_idx, 999)
    idxb = jnp.where(mask, bond_nei_idx[..., 0], 999)
    idxa_p = jnp.full((_B, _NP, _KP), 999, jnp.int32).at[:, :_N, :_K].set(idxa)
    idxb_p = jnp.full((_B, _NP, _KP), 999, jnp.int32).at[:, :_N, :_K].set(idxb)

    WaT = jnp.zeros((_H, _H), f32).at[:_AF, :].set(W_atom.T)
    W2aT = W_U2[:, :_H].T
    W2b_aug = jnp.concatenate([W_U2[:, _H:], b_U2[:, None]], axis=1)
    W2bT = jnp.zeros((_H, _H), f32).at[:_FB + 1, :].set(W2b_aug.T)
    W1aT = W_U1[:, :_H].T
    W1bT = W_U1[:, _H:].T
    bU1 = b_U1[None, :]

    wspec = pl.BlockSpec((_H, _H), lambda b: (0, 0))
    out = pl.pallas_call(
        _tc_body,
        grid=(_B,),
        in_specs=[
            pl.BlockSpec((1, _NP, _H), lambda b: (b, 0, 0)),
            pl.BlockSpec((1, _NP, _H), lambda b: (b, 0, 0)),
            pl.BlockSpec((1, _NP, _KP), lambda b: (b, 0, 0)),
            pl.BlockSpec((1, _NP, _KP), lambda b: (b, 0, 0)),
            wspec, wspec, wspec, wspec, wspec,
            pl.BlockSpec((1, _H), lambda b: (0, 0)),
        ],
        out_specs=pl.BlockSpec((1, _NP, _H), lambda b: (b, 0, 0)),
        out_shape=jax.ShapeDtypeStruct((_B, _NP, _H), f32),
    )(xp, bond_aug, idxa_p, idxb_p, WaT, W2aT, W2bT, W1aT, W1bT, bU1)
    return out[:, :_N, :]
